# Initial kernel scaffold; baseline (speedup 1.0000x reference)
import jax
import jax.numpy as jnp
from jax.experimental import pallas as pl


def kernel(x_nchw, embedding):
    raise NotImplementedError("write your pallas kernel here")


# speedup vs baseline: 1.1506x; 1.1506x over previous
import jax
import jax.numpy as jnp
from jax.experimental import pallas as pl
from jax.experimental.pallas import tpu as pltpu


def _vq_tile_kernel(emb_ref, embT_ref, e2_ref, x_ref, q_ref, hist_ref,
                    sse_ref, *, n_t):
    t = pl.program_id(1)

    x_t = x_ref[0]
    emb = emb_ref[...]
    K = emb.shape[0]
    TM = x_t.shape[1]

    x2 = jnp.sum(x_t * x_t, axis=0, keepdims=True)
    cross = jnp.dot(emb, x_t, preferred_element_type=jnp.float32)
    dist = x2 + e2_ref[...] - 2.0 * cross

    min_d = jnp.min(dist, axis=0, keepdims=True)
    k_iota = jax.lax.broadcasted_iota(jnp.int32, (K, TM), 0).astype(jnp.float32)
    cand = jnp.where(dist <= min_d, k_iota, jnp.float32(K))
    idx = jnp.min(cand, axis=0, keepdims=True)
    one_hot = (k_iota == idx).astype(jnp.float32)

    q_t = jnp.dot(embT_ref[...], one_hot, preferred_element_type=jnp.float32)
    q_ref[0] = q_t

    hist_t = jnp.sum(one_hot, axis=1, keepdims=True)
    diff = q_t - x_t
    sse_row = jnp.sum(diff * diff, axis=1, keepdims=True)
    sse_t = jnp.sum(sse_row, axis=0, keepdims=True)

    if n_t == 1:
        hist_ref[0] = hist_t
        sse_ref[0] = sse_t
    else:
        @pl.when(t == 0)
        def _init():
            hist_ref[...] = jnp.zeros_like(hist_ref)
            sse_ref[...] = jnp.zeros_like(sse_ref)

        hist_ref[0] = hist_ref[0] + hist_t
        sse_ref[0] = sse_ref[0] + sse_t


def _pick_tile(hw, target=2048):
    if hw <= target:
        return hw
    best = hw
    t = 128
    while t <= target:
        if hw % t == 0:
            best = t
        t += 128
    return best


def kernel(x_nchw, embedding, *, commitment_cost=0.25):
    x = x_nchw.astype(jnp.float32)
    B, D, H, W = x.shape
    K, D2 = embedding.shape
    assert D == D2, "embedding_dim mismatch"
    HW = H * W

    tm = _pick_tile(HW)
    n_t = HW // tm

    x_flat = x.reshape(B, D, HW)

    emb = embedding.astype(jnp.float32)
    embT = emb.T
    e2 = jnp.sum(emb * emb, axis=1, keepdims=True)

    flops = int(4 * B * HW * K * D)
    bytes_accessed = int(4 * (2 * B * HW * D + 2 * K * D + K + B * (K + 1)))

    import functools
    body = functools.partial(_vq_tile_kernel, n_t=n_t)

    q_flat, hist, sse = pl.pallas_call(
        body,
        out_shape=(
            jax.ShapeDtypeStruct((B, D, HW), jnp.float32),
            jax.ShapeDtypeStruct((B, K, 1), jnp.float32),
            jax.ShapeDtypeStruct((B, 1, 1), jnp.float32),
        ),
        grid_spec=pltpu.PrefetchScalarGridSpec(
            num_scalar_prefetch=0,
            grid=(B, n_t),
            in_specs=[
                pl.BlockSpec((K, D), lambda b, t: (0, 0)),
                pl.BlockSpec((D, K), lambda b, t: (0, 0)),
                pl.BlockSpec((K, 1), lambda b, t: (0, 0)),
                pl.BlockSpec((1, D, tm), lambda b, t: (b, 0, t)),
            ],
            out_specs=(
                pl.BlockSpec((1, D, tm), lambda b, t: (b, 0, t)),
                pl.BlockSpec((1, K, 1), lambda b, t: (b, 0, 0)),
                pl.BlockSpec((1, 1, 1), lambda b, t: (b, 0, 0)),
            ),
        ),
        compiler_params=pltpu.CompilerParams(
            dimension_semantics=("parallel", "arbitrary"),
            vmem_limit_bytes=64 * 1024 * 1024,
        ),
        cost_estimate=pl.CostEstimate(
            flops=flops, transcendentals=0, bytes_accessed=bytes_accessed),
    )(emb, embT, e2, x_flat)

    quantized = q_flat.reshape(B, D, H, W)
    mse = jnp.sum(sse) / (B * D * H * W)
    loss = (1.0 + commitment_cost) * mse
    index_histogram = hist[:, :, 0]
    return quantized, loss, index_histogram


# speedup vs baseline: 1.1830x; 1.0281x over previous
import functools

import jax
import jax.numpy as jnp
from jax.experimental import pallas as pl
from jax.experimental.pallas import tpu as pltpu


def _vq_tile_kernel(embm2_ref, embT_ref, e2_ref, kcol_ref, x_ref,
                    q_ref, hist_ref, sse_ref, *, n_t):
    t = pl.program_id(1)

    x_t = x_ref[0]
    K = embm2_ref.shape[0]

    x2 = jnp.sum(x_t * x_t, axis=0, keepdims=True)
    ncross2 = jnp.dot(embm2_ref[...], x_t,
                      preferred_element_type=jnp.float32)
    dist = (x2 + e2_ref[...]) + ncross2

    min_d = jnp.min(dist, axis=0, keepdims=True)
    kcol = kcol_ref[...]
    cand = jnp.where(dist <= min_d, kcol, jnp.float32(K))
    idx = jnp.min(cand, axis=0, keepdims=True)
    one_hot = (kcol == idx).astype(jnp.float32)

    q_t = jnp.dot(embT_ref[...], one_hot, preferred_element_type=jnp.float32)
    q_ref[0] = q_t

    hist_t = jnp.sum(one_hot, axis=1, keepdims=True)
    diff = q_t - x_t
    sse_row = jnp.sum(diff * diff, axis=1, keepdims=True)
    sse_t = jnp.sum(sse_row, axis=0, keepdims=True)

    if n_t == 1:
        hist_ref[0] = hist_t
        sse_ref[0] = sse_t
    else:
        @pl.when(t == 0)
        def _init():
            hist_ref[...] = jnp.zeros_like(hist_ref)
            sse_ref[...] = jnp.zeros_like(sse_ref)

        hist_ref[0] = hist_ref[0] + hist_t
        sse_ref[0] = sse_ref[0] + sse_t


def _pick_tile(hw, target=2048):
    if hw <= target:
        return hw
    best = hw
    t = 128
    while t <= target:
        if hw % t == 0:
            best = t
        t += 128
    return best


def kernel(x_nchw, embedding, *, commitment_cost=0.25):
    x = x_nchw.astype(jnp.float32)
    B, D, H, W = x.shape
    K, D2 = embedding.shape
    assert D == D2, "embedding_dim mismatch"
    HW = H * W

    tm = _pick_tile(HW)
    n_t = HW // tm

    x_flat = x.reshape(B, D, HW)

    emb = embedding.astype(jnp.float32)
    embm2 = -2.0 * emb
    embT = emb.T
    e2 = jnp.sum(emb * emb, axis=1, keepdims=True)
    kcol = jnp.arange(K, dtype=jnp.float32)[:, None]

    flops = int(4 * B * HW * K * D)
    bytes_accessed = int(4 * (2 * B * HW * D + 2 * K * D + K + B * (K + 1)))

    body = functools.partial(_vq_tile_kernel, n_t=n_t)

    q_flat, hist, sse = pl.pallas_call(
        body,
        out_shape=(
            jax.ShapeDtypeStruct((B, D, HW), jnp.float32),
            jax.ShapeDtypeStruct((B, K, 1), jnp.float32),
            jax.ShapeDtypeStruct((B, 1, 1), jnp.float32),
        ),
        grid_spec=pltpu.PrefetchScalarGridSpec(
            num_scalar_prefetch=0,
            grid=(B, n_t),
            in_specs=[
                pl.BlockSpec((K, D), lambda b, t: (0, 0)),
                pl.BlockSpec((D, K), lambda b, t: (0, 0)),
                pl.BlockSpec((K, 1), lambda b, t: (0, 0)),
                pl.BlockSpec((K, 1), lambda b, t: (0, 0)),
                pl.BlockSpec((1, D, tm), lambda b, t: (b, 0, t)),
            ],
            out_specs=(
                pl.BlockSpec((1, D, tm), lambda b, t: (b, 0, t)),
                pl.BlockSpec((1, K, 1), lambda b, t: (b, 0, 0)),
                pl.BlockSpec((1, 1, 1), lambda b, t: (b, 0, 0)),
            ),
        ),
        compiler_params=pltpu.CompilerParams(
            dimension_semantics=("parallel", "arbitrary"),
            vmem_limit_bytes=64 * 1024 * 1024,
        ),
        cost_estimate=pl.CostEstimate(
            flops=flops, transcendentals=0, bytes_accessed=bytes_accessed),
    )(embm2, embT, e2, kcol, x_flat)

    quantized = q_flat.reshape(B, D, H, W)
    mse = jnp.sum(sse) / (B * D * H * W)
    loss = (1.0 + commitment_cost) * mse
    index_histogram = hist[:, :, 0]
    return quantized, loss, index_histogram


# speedup vs baseline: 1.2023x; 1.0164x over previous
import functools

import jax
import jax.numpy as jnp
from jax.experimental import pallas as pl
from jax.experimental.pallas import tpu as pltpu


def _vq_tile_kernel(embm2_ref, embT_ref, e2_ref, kcol_ref, x_ref,
                    q_ref, hist_ref, sse_ref, *, n_t):
    t = pl.program_id(1)

    x_t = x_ref[0]
    K = embm2_ref.shape[0]

    x2 = jnp.sum(x_t * x_t, axis=0, keepdims=True)
    ncross2 = jnp.dot(embm2_ref[...], x_t,
                      preferred_element_type=jnp.float32)
    dist = (x2 + e2_ref[...]) + ncross2

    min_d = jnp.min(dist, axis=0, keepdims=True)
    kcol = kcol_ref[...]
    cand = jnp.where(dist <= min_d, kcol, jnp.float32(K))
    idx = jnp.min(cand, axis=0, keepdims=True)
    one_hot = (kcol == idx).astype(jnp.float32)

    q_t = jnp.dot(embT_ref[...], one_hot, preferred_element_type=jnp.float32)
    q_ref[0] = q_t

    hist_t = jnp.sum(one_hot, axis=1, keepdims=True)
    diff = q_t - x_t
    sse_row = jnp.sum(diff * diff, axis=1, keepdims=True)
    sse_t = jnp.sum(sse_row, axis=0, keepdims=True)

    if n_t == 1:
        hist_ref[0] = hist_t
        sse_ref[0] = sse_t
    else:
        @pl.when(t == 0)
        def _init():
            hist_ref[...] = jnp.zeros_like(hist_ref)
            sse_ref[...] = jnp.zeros_like(sse_ref)

        hist_ref[0] = hist_ref[0] + hist_t
        sse_ref[0] = sse_ref[0] + sse_t


def _pick_tile(hw, target=4096):
    if hw <= target:
        return hw
    best = hw
    t = 128
    while t <= target:
        if hw % t == 0:
            best = t
        t += 128
    return best


def kernel(x_nchw, embedding, *, commitment_cost=0.25):
    x = x_nchw.astype(jnp.float32)
    B, D, H, W = x.shape
    K, D2 = embedding.shape
    assert D == D2, "embedding_dim mismatch"
    HW = H * W

    tm = _pick_tile(HW)
    n_t = HW // tm

    x_flat = x.reshape(B, D, HW)

    emb = embedding.astype(jnp.float32)
    embm2 = -2.0 * emb
    embT = emb.T
    e2 = jnp.sum(emb * emb, axis=1, keepdims=True)
    kcol = jnp.arange(K, dtype=jnp.float32)[:, None]

    flops = int(4 * B * HW * K * D)
    bytes_accessed = int(4 * (2 * B * HW * D + 2 * K * D + K + B * (K + 1)))

    body = functools.partial(_vq_tile_kernel, n_t=n_t)

    q_flat, hist, sse = pl.pallas_call(
        body,
        out_shape=(
            jax.ShapeDtypeStruct((B, D, HW), jnp.float32),
            jax.ShapeDtypeStruct((B, K, 1), jnp.float32),
            jax.ShapeDtypeStruct((B, 1, 1), jnp.float32),
        ),
        grid_spec=pltpu.PrefetchScalarGridSpec(
            num_scalar_prefetch=0,
            grid=(B, n_t),
            in_specs=[
                pl.BlockSpec((K, D), lambda b, t: (0, 0)),
                pl.BlockSpec((D, K), lambda b, t: (0, 0)),
                pl.BlockSpec((K, 1), lambda b, t: (0, 0)),
                pl.BlockSpec((K, 1), lambda b, t: (0, 0)),
                pl.BlockSpec((1, D, tm), lambda b, t: (b, 0, t)),
            ],
            out_specs=(
                pl.BlockSpec((1, D, tm), lambda b, t: (b, 0, t)),
                pl.BlockSpec((1, K, 1), lambda b, t: (b, 0, 0)),
                pl.BlockSpec((1, 1, 1), lambda b, t: (b, 0, 0)),
            ),
        ),
        compiler_params=pltpu.CompilerParams(
            dimension_semantics=("parallel", "arbitrary"),
            vmem_limit_bytes=64 * 1024 * 1024,
        ),
        cost_estimate=pl.CostEstimate(
            flops=flops, transcendentals=0, bytes_accessed=bytes_accessed),
    )(embm2, embT, e2, kcol, x_flat)

    quantized = q_flat.reshape(B, D, H, W)
    mse = jnp.sum(sse) / (B * D * H * W)
    loss = (1.0 + commitment_cost) * mse
    index_histogram = hist[:, :, 0]
    return quantized, loss, index_histogram
